# baseline (speedup 1.0000x reference)
import functools

import jax
import jax.numpy as jnp
from jax import lax
from jax.experimental import pallas as pl
from jax.experimental.pallas import tpu as pltpu
from jax.experimental.pallas import tpu_sc as plsc

ENT_NUM = 1000000
REL_NUM = 1000
EMB_DIM = 64
BATCH = 16384
PACK = 128
ENT_P = ENT_NUM // 2
REL_P = REL_NUM // 2

NC = 2
NS = 16
NW = NC * NS
B_PER_W = BATCH // NW
L = 16
CHUNK = 128
NCHUNK = B_PER_W // CHUNK
NCH = EMB_DIM // L


def _body(heads_hbm, rels_hbm, tails_hbm, ent2_hbm, rel2_hbm, out_hbm,
          hidx, ridx, tidx, hpk, rpk, tpk, hbuf, rbuf, tbuf, outv, sem):
    wid = lax.axis_index("s") * NC + lax.axis_index("c")
    base = pl.multiple_of(wid * B_PER_W, B_PER_W)

    pltpu.sync_copy(heads_hbm.at[pl.ds(base, B_PER_W)], hidx)
    pltpu.sync_copy(rels_hbm.at[pl.ds(base, B_PER_W)], ridx)
    pltpu.sync_copy(tails_hbm.at[pl.ds(base, B_PER_W)], tidx)

    def shift_blk(v, _):
        sl = pl.ds(pl.multiple_of(v * L, L), L)
        hpk[sl] = lax.shift_right_logical(hidx[sl], 1)
        rpk[sl] = lax.shift_right_logical(ridx[sl], 1)
        tpk[sl] = lax.shift_right_logical(tidx[sl], 1)
        return 0

    lax.fori_loop(0, B_PER_W // L, shift_blk, 0)

    lanes = lax.iota(jnp.int32, L)

    def chunk(c, _):
        cbase = c * CHUNK
        sl = pl.ds(cbase, CHUNK)
        cps = [pltpu.async_copy(ent2_hbm.at[hpk.at[sl]], hbuf, sem),
               pltpu.async_copy(ent2_hbm.at[tpk.at[sl]], tbuf, sem),
               pltpu.async_copy(rel2_hbm.at[rpk.at[sl]], rbuf, sem)]
        for cp in cps:
            cp.wait()
        for g in range(CHUNK // L):
            gsl = pl.ds(pl.multiple_of(cbase + g * L, L), L)
            ho_v = (hidx[gsl] & 1) * EMB_DIM
            ro_v = (ridx[gsl] & 1) * EMB_DIM
            to_v = (tidx[gsl] & 1) * EMB_DIM
            acc16 = jnp.zeros((L,), jnp.float32)
            for k in range(L):
                r = g * L + k
                ho = pl.multiple_of(ho_v[k], EMB_DIM)
                ro = pl.multiple_of(ro_v[k], EMB_DIM)
                to = pl.multiple_of(to_v[k], EMB_DIM)
                acc = (hbuf[r, pl.ds(ho, L)] * rbuf[r, pl.ds(ro, L)]
                       * tbuf[r, pl.ds(to, L)])
                for cc in range(1, NCH):
                    acc = acc + (hbuf[r, pl.ds(ho + cc * L, L)]
                                 * rbuf[r, pl.ds(ro + cc * L, L)]
                                 * tbuf[r, pl.ds(to + cc * L, L)])
                s = lax.reduce_sum(acc, axes=(0,))
                acc16 = jnp.where(lanes == k, s, acc16)
            outv[pl.ds(pl.multiple_of(cbase + g * L, L), L)] = acc16
        return 0

    lax.fori_loop(0, NCHUNK, chunk, 0)

    pltpu.sync_copy(outv, out_hbm.at[pl.ds(base, B_PER_W)])


@jax.jit
def _distmult(heads, rels, tails, ent2, rel2):
    mesh = plsc.VectorSubcoreMesh(core_axis_name="c", subcore_axis_name="s")
    return pl.kernel(
        _body,
        out_type=jax.ShapeDtypeStruct((BATCH,), jnp.float32),
        mesh=mesh,
        compiler_params=pltpu.CompilerParams(
            needs_layout_passes=False, use_tc_tiling_on_sc=True),
        scratch_types=[
            pltpu.VMEM((B_PER_W,), jnp.int32),
            pltpu.VMEM((B_PER_W,), jnp.int32),
            pltpu.VMEM((B_PER_W,), jnp.int32),
            pltpu.VMEM((B_PER_W,), jnp.int32),
            pltpu.VMEM((B_PER_W,), jnp.int32),
            pltpu.VMEM((B_PER_W,), jnp.int32),
            pltpu.VMEM((CHUNK, PACK), jnp.float32),
            pltpu.VMEM((CHUNK, PACK), jnp.float32),
            pltpu.VMEM((CHUNK, PACK), jnp.float32),
            pltpu.VMEM((B_PER_W,), jnp.float32),
            pltpu.SemaphoreType.DMA,
        ],
    )(heads, rels, tails, ent2, rel2)


def kernel(heads, rels, tails, ent_embeds, rel_embeds):
    ent2 = ent_embeds.reshape(ENT_P, PACK)
    rel2 = rel_embeds.reshape(REL_P, PACK)
    return _distmult(heads.astype(jnp.int32), rels.astype(jnp.int32),
                     tails.astype(jnp.int32), ent2, rel2)
